# baseline (speedup 1.0000x reference)
import functools

import jax
import jax.numpy as jnp
from jax.experimental import pallas as pl
from jax.experimental.pallas import tpu as pltpu

_SLOPE = 0.02
_EPS = 1e-5


def _conv_stats_kernel(x_ref, w_ref, y_ref, s1_ref, s2_ref, *, H, W):
    NBL = x_ref.shape[1]
    _, C, HW = y_ref.shape
    CHT = min(64, NBL)
    CH = min(16, NBL)

    p = jax.lax.broadcasted_iota(jnp.int32, (1, 1, HW), 2)
    w_of_p = p % W
    w9 = w_ref[...]

    s1 = jnp.zeros((C, HW), jnp.float32)
    s2 = jnp.zeros((C, HW), jnp.float32)
    for t0 in range(0, NBL, CHT):
        xv = x_ref[:, t0:t0 + CHT].astype(jnp.bfloat16)
        av = jnp.where(xv >= 0, xv, xv * _SLOPE)
        a_t = jnp.transpose(av).reshape(CHT, C, HW)

        for i0 in range(0, CHT, CH):
            a = a_t[i0:i0 + CH]

            segs = []
            for kd in range(3):
                for wd in range(3):
                    kh, kw = 2 - kd, 2 - wd
                    d = (kh - 1) * W + (kw - 1)
                    if d > 0:
                        s = jnp.concatenate(
                            [a[:, :, d:],
                             jnp.zeros((CH, C, d), a.dtype)], axis=2)
                    elif d < 0:
                        s = jnp.concatenate(
                            [jnp.zeros((CH, C, -d), a.dtype),
                             a[:, :, :d]], axis=2)
                    else:
                        s = a
                    m = jnp.ones((1, 1, HW), dtype=jnp.bool_)
                    if kh == 0:
                        m = m & (p >= W)
                    elif kh == 2:
                        m = m & (p < HW - W)
                    if kw == 0:
                        m = m & (w_of_p >= 1)
                    elif kw == 2:
                        m = m & (w_of_p < W - 1)
                    segs.append(jnp.where(m, s, jnp.zeros_like(s)))
            a9 = jnp.concatenate(segs, axis=1)

            for i in range(CH):
                y = jax.lax.dot_general(
                    w9, a9[i], (((0,), (0,)), ((), ())),
                    preferred_element_type=jnp.float32)
                y_ref[t0 + i0 + i] = y.astype(y_ref.dtype)
                s1 = s1 + y
                s2 = s2 + y * y
    s1_ref[...] = jnp.sum(s1, axis=1, keepdims=True).reshape(1, C, 1)
    s2_ref[...] = jnp.sum(s2, axis=1, keepdims=True).reshape(1, C, 1)


def _bn_apply_kernel(y_ref, scale_ref, shift_ref, out_ref):
    NBL, C, HW = y_ref.shape
    yt = jnp.transpose(y_ref[...].reshape(NBL, C * HW))
    out_ref[...] = (yt.astype(jnp.float32) * scale_ref[...]
                    + shift_ref[...])


@jax.jit
def _forward(x_nchw, w_deconv, gamma, beta):
    N, C, H, W = x_nchw.shape
    HW = H * W
    CHW = C * HW
    NBL = 128
    while N % NBL:
        NBL //= 2
    GL = N // NBL

    xv = jnp.transpose(x_nchw, (1, 2, 3, 0)).reshape(CHW, N)
    w9 = jnp.transpose(w_deconv, (2, 3, 0, 1)).reshape(9 * C, C)
    w9 = w9.astype(jnp.bfloat16)

    cparams = pltpu.CompilerParams(
        dimension_semantics=("parallel",),
        vmem_limit_bytes=64 * 1024 * 1024,
    )

    conv_kernel = functools.partial(_conv_stats_kernel, H=H, W=W)
    y, s1, s2 = pl.pallas_call(
        conv_kernel,
        grid=(GL,),
        in_specs=[
            pl.BlockSpec((CHW, NBL), lambda g: (0, g)),
            pl.BlockSpec((9 * C, C), lambda g: (0, 0)),
        ],
        out_specs=(
            pl.BlockSpec((NBL, C, HW), lambda g: (g, 0, 0)),
            pl.BlockSpec((1, C, 1), lambda g: (g, 0, 0)),
            pl.BlockSpec((1, C, 1), lambda g: (g, 0, 0)),
        ),
        out_shape=(
            jax.ShapeDtypeStruct((N, C, HW), jnp.bfloat16),
            jax.ShapeDtypeStruct((GL, C, 1), jnp.float32),
            jax.ShapeDtypeStruct((GL, C, 1), jnp.float32),
        ),
        compiler_params=cparams,
    )(xv, w9)

    m_total = float(N * H * W)
    sum_c = jnp.sum(s1, axis=(0, 2))
    sq_c = jnp.sum(s2, axis=(0, 2))
    mean = sum_c / m_total
    var = jnp.maximum(sq_c / m_total - mean * mean, 0.0)
    inv = jax.lax.rsqrt(var + _EPS)
    scale_c = gamma.astype(jnp.float32) * inv
    shift_c = beta.astype(jnp.float32) - mean * scale_c
    scale_r = jnp.broadcast_to(scale_c[:, None, None], (C, HW, 1))
    scale_r = scale_r.reshape(CHW, 1)
    shift_r = jnp.broadcast_to(shift_c[:, None, None], (C, HW, 1))
    shift_r = shift_r.reshape(CHW, 1)

    out_v = pl.pallas_call(
        _bn_apply_kernel,
        grid=(GL,),
        in_specs=[
            pl.BlockSpec((NBL, C, HW), lambda g: (g, 0, 0)),
            pl.BlockSpec((CHW, 1), lambda g: (0, 0)),
            pl.BlockSpec((CHW, 1), lambda g: (0, 0)),
        ],
        out_specs=pl.BlockSpec((CHW, NBL), lambda g: (0, g)),
        out_shape=jax.ShapeDtypeStruct((CHW, N), jnp.float32),
        compiler_params=cparams,
    )(y, scale_r, shift_r)

    return jnp.transpose(out_v.reshape(C, H, W, N), (3, 0, 1, 2))


def kernel(x_nchw, w_deconv, gamma, beta):
    return _forward(x_nchw, w_deconv, gamma, beta)


# speedup vs baseline: 1.3948x; 1.3948x over previous
import functools

import jax
import jax.numpy as jnp
from jax.experimental import pallas as pl
from jax.experimental.pallas import tpu as pltpu

_SLOPE = 0.02
_EPS = 1e-5


def _conv_stats_kernel(x_ref, w_ref, y_ref, s1_ref, s2_ref, *, H, W):
    NBL = x_ref.shape[1]
    _, C, HW = y_ref.shape
    CHT = NBL
    CH = min(16, NBL)

    p = jax.lax.broadcasted_iota(jnp.int32, (1, 1, HW), 2)
    w_of_p = p % W
    w9 = w_ref[...]

    s1 = jnp.zeros((C, HW), jnp.float32)
    s2 = jnp.zeros((C, HW), jnp.float32)
    for t0 in range(0, NBL, CHT):
        xv = x_ref[:, t0:t0 + CHT].astype(jnp.bfloat16)
        av = jnp.where(xv >= 0, xv, xv * _SLOPE)
        a_t = jnp.transpose(av).reshape(CHT, C, HW)

        for i0 in range(0, CHT, CH):
            a = a_t[i0:i0 + CH]

            segs = []
            for kd in range(3):
                for wd in range(3):
                    kh, kw = 2 - kd, 2 - wd
                    d = (kh - 1) * W + (kw - 1)
                    if d > 0:
                        s = jnp.concatenate(
                            [a[:, :, d:],
                             jnp.zeros((CH, C, d), a.dtype)], axis=2)
                    elif d < 0:
                        s = jnp.concatenate(
                            [jnp.zeros((CH, C, -d), a.dtype),
                             a[:, :, :d]], axis=2)
                    else:
                        s = a
                    m = jnp.ones((1, 1, HW), dtype=jnp.bool_)
                    if kh == 0:
                        m = m & (p >= W)
                    elif kh == 2:
                        m = m & (p < HW - W)
                    if kw == 0:
                        m = m & (w_of_p >= 1)
                    elif kw == 2:
                        m = m & (w_of_p < W - 1)
                    segs.append(jnp.where(m, s, jnp.zeros_like(s)))
            a9 = jnp.concatenate(segs, axis=1)

            for i in range(CH):
                y = jax.lax.dot_general(
                    w9, a9[i], (((0,), (0,)), ((), ())),
                    preferred_element_type=jnp.float32)
                y_ref[t0 + i0 + i] = y.astype(y_ref.dtype)
                s1 = s1 + y
                s2 = s2 + y * y
    s1_ref[...] = jnp.sum(s1, axis=1, keepdims=True).reshape(1, C, 1)
    s2_ref[...] = jnp.sum(s2, axis=1, keepdims=True).reshape(1, C, 1)


def _bn_apply_kernel(y_ref, scale_ref, shift_ref, out_ref):
    NBL, C, HW = y_ref.shape
    o = (y_ref[...].astype(jnp.float32) * scale_ref[...]
         + shift_ref[...]).astype(jnp.bfloat16)
    out_ref[...] = jnp.transpose(o.reshape(NBL, C * HW)).astype(jnp.float32)


@jax.jit
def _forward(x_nchw, w_deconv, gamma, beta):
    N, C, H, W = x_nchw.shape
    HW = H * W
    CHW = C * HW
    NBL = 128
    while N % NBL:
        NBL //= 2
    GL = N // NBL

    xv = jnp.transpose(x_nchw, (1, 2, 3, 0)).reshape(CHW, N)
    w9 = jnp.transpose(w_deconv, (2, 3, 0, 1)).reshape(9 * C, C)
    w9 = w9.astype(jnp.bfloat16)

    cparams = pltpu.CompilerParams(
        dimension_semantics=("parallel",),
        vmem_limit_bytes=64 * 1024 * 1024,
    )

    conv_kernel = functools.partial(_conv_stats_kernel, H=H, W=W)
    y, s1, s2 = pl.pallas_call(
        conv_kernel,
        grid=(GL,),
        in_specs=[
            pl.BlockSpec((CHW, NBL), lambda g: (0, g)),
            pl.BlockSpec((9 * C, C), lambda g: (0, 0)),
        ],
        out_specs=(
            pl.BlockSpec((NBL, C, HW), lambda g: (g, 0, 0)),
            pl.BlockSpec((1, C, 1), lambda g: (g, 0, 0)),
            pl.BlockSpec((1, C, 1), lambda g: (g, 0, 0)),
        ),
        out_shape=(
            jax.ShapeDtypeStruct((N, C, HW), jnp.bfloat16),
            jax.ShapeDtypeStruct((GL, C, 1), jnp.float32),
            jax.ShapeDtypeStruct((GL, C, 1), jnp.float32),
        ),
        compiler_params=cparams,
    )(xv, w9)

    m_total = float(N * H * W)
    sum_c = jnp.sum(s1, axis=(0, 2))
    sq_c = jnp.sum(s2, axis=(0, 2))
    mean = sum_c / m_total
    var = jnp.maximum(sq_c / m_total - mean * mean, 0.0)
    inv = jax.lax.rsqrt(var + _EPS)
    scale_c = gamma.astype(jnp.float32) * inv
    shift_c = beta.astype(jnp.float32) - mean * scale_c
    scale_r = jnp.broadcast_to(scale_c[:, None], (C, HW))
    shift_r = jnp.broadcast_to(shift_c[:, None], (C, HW))

    out_v = pl.pallas_call(
        _bn_apply_kernel,
        grid=(GL,),
        in_specs=[
            pl.BlockSpec((NBL, C, HW), lambda g: (g, 0, 0)),
            pl.BlockSpec((C, HW), lambda g: (0, 0)),
            pl.BlockSpec((C, HW), lambda g: (0, 0)),
        ],
        out_specs=pl.BlockSpec((CHW, NBL), lambda g: (0, g)),
        out_shape=jax.ShapeDtypeStruct((CHW, N), jnp.float32),
        compiler_params=cparams,
    )(y, scale_r, shift_r)

    return jnp.transpose(out_v.reshape(C, H, W, N), (3, 0, 1, 2))


def kernel(x_nchw, w_deconv, gamma, beta):
    return _forward(x_nchw, w_deconv, gamma, beta)


# speedup vs baseline: 1.4036x; 1.0063x over previous
import functools

import jax
import jax.numpy as jnp
from jax.experimental import pallas as pl
from jax.experimental.pallas import tpu as pltpu

_SLOPE = 0.02
_EPS = 1e-5


def _conv_stats_kernel(x_ref, w_ref, y_ref, s1_ref, s2_ref, *, H, W):
    NBL = x_ref.shape[1]
    _, C, HW = y_ref.shape
    CHT = NBL
    CH = min(32, NBL)

    p = jax.lax.broadcasted_iota(jnp.int32, (1, 1, HW), 2)
    w_of_p = p % W
    w9 = w_ref[...]

    s1 = jnp.zeros((C, HW), jnp.float32)
    s2 = jnp.zeros((C, HW), jnp.float32)
    for t0 in range(0, NBL, CHT):
        xv = x_ref[:, t0:t0 + CHT].astype(jnp.bfloat16)
        av = jnp.where(xv >= 0, xv, xv * _SLOPE)
        a_t = jnp.transpose(av).reshape(CHT, C, HW)

        for i0 in range(0, CHT, CH):
            a = a_t[i0:i0 + CH]

            segs = []
            for kd in range(3):
                for wd in range(3):
                    kh, kw = 2 - kd, 2 - wd
                    d = (kh - 1) * W + (kw - 1)
                    if d > 0:
                        s = jnp.concatenate(
                            [a[:, :, d:],
                             jnp.zeros((CH, C, d), a.dtype)], axis=2)
                    elif d < 0:
                        s = jnp.concatenate(
                            [jnp.zeros((CH, C, -d), a.dtype),
                             a[:, :, :d]], axis=2)
                    else:
                        s = a
                    m = jnp.ones((1, 1, HW), dtype=jnp.bool_)
                    if kh == 0:
                        m = m & (p >= W)
                    elif kh == 2:
                        m = m & (p < HW - W)
                    if kw == 0:
                        m = m & (w_of_p >= 1)
                    elif kw == 2:
                        m = m & (w_of_p < W - 1)
                    segs.append(jnp.where(m, s, jnp.zeros_like(s)))
            a9 = jnp.concatenate(segs, axis=1)

            for i in range(CH):
                y = jax.lax.dot_general(
                    w9, a9[i], (((0,), (0,)), ((), ())),
                    preferred_element_type=jnp.float32)
                y_ref[t0 + i0 + i] = y.astype(y_ref.dtype)
                s1 = s1 + y
                s2 = s2 + y * y
    s1_ref[...] = jnp.sum(s1, axis=1, keepdims=True).reshape(1, C, 1)
    s2_ref[...] = jnp.sum(s2, axis=1, keepdims=True).reshape(1, C, 1)


def _bn_apply_kernel(y_ref, scale_ref, shift_ref, out_ref):
    NBL, C, HW = y_ref.shape
    o = (y_ref[...].astype(jnp.float32) * scale_ref[...]
         + shift_ref[...]).astype(jnp.bfloat16)
    out_ref[...] = jnp.transpose(o.reshape(NBL, C * HW)).astype(jnp.float32)


@jax.jit
def _forward(x_nchw, w_deconv, gamma, beta):
    N, C, H, W = x_nchw.shape
    HW = H * W
    CHW = C * HW
    NBL = 128
    while N % NBL:
        NBL //= 2
    GL = N // NBL
    NB2 = 128
    while N % NB2:
        NB2 //= 2
    G2 = N // NB2

    xv = jnp.transpose(x_nchw, (1, 2, 3, 0)).reshape(CHW, N)
    w9 = jnp.transpose(w_deconv, (2, 3, 0, 1)).reshape(9 * C, C)
    w9 = w9.astype(jnp.bfloat16)

    cparams = pltpu.CompilerParams(
        dimension_semantics=("parallel",),
        vmem_limit_bytes=64 * 1024 * 1024,
    )

    conv_kernel = functools.partial(_conv_stats_kernel, H=H, W=W)
    y, s1, s2 = pl.pallas_call(
        conv_kernel,
        grid=(GL,),
        in_specs=[
            pl.BlockSpec((CHW, NBL), lambda g: (0, g)),
            pl.BlockSpec((9 * C, C), lambda g: (0, 0)),
        ],
        out_specs=(
            pl.BlockSpec((NBL, C, HW), lambda g: (g, 0, 0)),
            pl.BlockSpec((1, C, 1), lambda g: (g, 0, 0)),
            pl.BlockSpec((1, C, 1), lambda g: (g, 0, 0)),
        ),
        out_shape=(
            jax.ShapeDtypeStruct((N, C, HW), jnp.bfloat16),
            jax.ShapeDtypeStruct((GL, C, 1), jnp.float32),
            jax.ShapeDtypeStruct((GL, C, 1), jnp.float32),
        ),
        compiler_params=cparams,
    )(xv, w9)

    m_total = float(N * H * W)
    sum_c = jnp.sum(s1, axis=(0, 2))
    sq_c = jnp.sum(s2, axis=(0, 2))
    mean = sum_c / m_total
    var = jnp.maximum(sq_c / m_total - mean * mean, 0.0)
    inv = jax.lax.rsqrt(var + _EPS)
    scale_c = gamma.astype(jnp.float32) * inv
    shift_c = beta.astype(jnp.float32) - mean * scale_c
    scale_r = jnp.broadcast_to(scale_c[:, None], (C, HW))
    shift_r = jnp.broadcast_to(shift_c[:, None], (C, HW))

    out_v = pl.pallas_call(
        _bn_apply_kernel,
        grid=(G2,),
        in_specs=[
            pl.BlockSpec((NB2, C, HW), lambda g: (g, 0, 0)),
            pl.BlockSpec((C, HW), lambda g: (0, 0)),
            pl.BlockSpec((C, HW), lambda g: (0, 0)),
        ],
        out_specs=pl.BlockSpec((CHW, NB2), lambda g: (0, g)),
        out_shape=jax.ShapeDtypeStruct((CHW, N), jnp.float32),
        compiler_params=cparams,
    )(y, scale_r, shift_r)

    return jnp.transpose(out_v.reshape(C, H, W, N), (3, 0, 1, 2))


def kernel(x_nchw, w_deconv, gamma, beta):
    return _forward(x_nchw, w_deconv, gamma, beta)
